# baseline (speedup 1.0000x reference)
import functools

import jax
import jax.numpy as jnp
from jax import lax
from jax.experimental import pallas as pl
from jax.experimental.pallas import tpu as pltpu
from jax.experimental.pallas import tpu_sc as plsc

_B = 16384
_K = 16
_DIM = 64
_AW = 8
_CH = 128


def _make_sc_gather(nc, ns, nl):
    nw = nc * ns
    bpw = _B // nw
    nch = bpw // _CH
    mesh = plsc.VectorSubcoreMesh(core_axis_name="c", subcore_axis_name="s")

    @functools.partial(
        pl.kernel,
        out_type=(
            jax.ShapeDtypeStruct((_B, _AW), jnp.float32),
            jax.ShapeDtypeStruct((_B, _K), jnp.float32),
            jax.ShapeDtypeStruct((_B, _K), jnp.float32),
        ),
        mesh=mesh,
        scratch_types=[
            pltpu.VMEM((nch, _CH), jnp.int32),
            pltpu.VMEM((nch, _CH), jnp.int32),
            pltpu.VMEM((bpw, _AW), jnp.float32),
            pltpu.VMEM((bpw, _K), jnp.float32),
            pltpu.VMEM((bpw, _K), jnp.float32),
            pltpu.SemaphoreType.DMA,
        ],
        compiler_params=pltpu.CompilerParams(use_tc_tiling_on_sc=False),
    )
    def gather(idx_hbm, a_hbm, A_hbm, P_hbm, a_out, A_out, P_out,
               idx_v, idx8_v, a_v, A_v, P_v, sem):
        wid = lax.axis_index("s") * nc + lax.axis_index("c")
        base = wid * bpw
        pltpu.sync_copy(idx_hbm.at[pl.ds(wid * nch, nch)], idx_v)
        for j in range(nch):
            for t in range(_CH // nl):
                sl = pl.ds(t * nl, nl)
                idx8_v[j, sl] = idx_v[j, sl] >> 3
        copies = []
        for j in range(nch):
            sl = pl.ds(j * _CH, _CH)
            copies.append(pltpu.async_copy(a_hbm.at[idx8_v.at[j]], a_v.at[sl], sem))
            copies.append(pltpu.async_copy(A_hbm.at[idx_v.at[j]], A_v.at[sl], sem))
            copies.append(pltpu.async_copy(P_hbm.at[idx_v.at[j]], P_v.at[sl], sem))
        for c in copies:
            c.wait()
        pltpu.sync_copy(a_v, a_out.at[pl.ds(base, bpw)])
        pltpu.sync_copy(A_v, A_out.at[pl.ds(base, bpw)])
        pltpu.sync_copy(P_v, P_out.at[pl.ds(base, bpw)])

    return gather


_BLK = 2048


def _dense_body(rel_ref, tau_ref, ga8_ref, gA_ref, gP_ref, omega_ref, w0_ref,
                wk_ref, b_ref, m_ref, sum_ref):
    i = pl.program_id(0)
    tau = tau_ref[:]
    lane = lax.rem(rel_ref[:], 8)
    sel = lane == lax.broadcasted_iota(jnp.int32, (1, _AW), 1)
    a = jnp.sum(jnp.where(sel, ga8_ref[:], 0.0), axis=1, keepdims=True)
    phase = tau * omega_ref[:] + gP_ref[:]
    zp = gA_ref[:] * jnp.sin(phase)
    acc = jnp.dot(zp, wk_ref[:], preferred_element_type=jnp.float32)
    m = jnp.tanh(acc + (a * tau) * w0_ref[:] + b_ref[:])
    m_ref[:] = m
    colsum = jnp.sum(m, axis=0, keepdims=True)

    @pl.when(i == 0)
    def _init():
        sum_ref[:] = colsum

    @pl.when(i != 0)
    def _acc():
        sum_ref[:] += colsum


def _sub_mean_body(m_ref, sum_ref, out_ref):
    out_ref[:] = m_ref[:] - sum_ref[:] * (1.0 / _B)


def kernel(rel_id, tau, a_r, A_r, P_r, omega, W_proj, b_proj):
    info = plsc.get_sparse_core_info()
    gather = _make_sc_gather(info.num_cores, info.num_subcores, info.num_lanes)
    ga8, gA, gP = gather(rel_id.reshape(-1, _CH), a_r.reshape(-1, _AW), A_r, P_r)
    nb = _B // _BLK
    row_spec = lambda w: pl.BlockSpec((_BLK, w), lambda i: (i, 0))
    rep_spec = lambda h, w: pl.BlockSpec((h, w), lambda i: (0, 0))
    m, colsum = pl.pallas_call(
        _dense_body,
        grid=(nb,),
        in_specs=[
            row_spec(1), row_spec(1), row_spec(_AW), row_spec(_K), row_spec(_K),
            rep_spec(1, _K), rep_spec(1, _DIM), rep_spec(_K, _DIM),
            rep_spec(1, _DIM),
        ],
        out_specs=[row_spec(_DIM), rep_spec(1, _DIM)],
        out_shape=[
            jax.ShapeDtypeStruct((_B, _DIM), jnp.float32),
            jax.ShapeDtypeStruct((1, _DIM), jnp.float32),
        ],
    )(rel_id.reshape(-1, 1), tau.reshape(-1, 1), ga8, gA, gP,
      omega.reshape(1, -1), W_proj[:, 0].reshape(1, -1), W_proj[:, 1:].T,
      b_proj.reshape(1, -1))
    return pl.pallas_call(
        _sub_mean_body,
        grid=(nb,),
        in_specs=[row_spec(_DIM), rep_spec(1, _DIM)],
        out_specs=row_spec(_DIM),
        out_shape=jax.ShapeDtypeStruct((_B, _DIM), jnp.float32),
    )(m, colsum)


# speedup vs baseline: 1.3371x; 1.3371x over previous
import functools

import jax
import jax.numpy as jnp
from jax import lax
from jax.experimental import pallas as pl
from jax.experimental.pallas import tpu as pltpu
from jax.experimental.pallas import tpu_sc as plsc

_B = 16384
_K = 16
_DIM = 64
_CH = 128
_APAD = 782
_PK = 8
_M = _B * _K // 128


def _make_sc_gather(nc, ns, nl):
    nw = nc * ns
    bpw = _B // nw
    nch = bpw // _CH
    mesh = plsc.VectorSubcoreMesh(core_axis_name="c", subcore_axis_name="s")

    @functools.partial(
        pl.kernel,
        out_type=(
            jax.ShapeDtypeStruct((_B, _K), jnp.float32),
            jax.ShapeDtypeStruct((_B, _K), jnp.float32),
            jax.ShapeDtypeStruct((_B, _K), jnp.float32),
        ),
        mesh=mesh,
        scratch_types=[
            pltpu.VMEM((bpw,), jnp.int32),
            pltpu.VMEM((bpw,), jnp.int32),
            pltpu.VMEM((bpw,), jnp.float32),
            pltpu.VMEM((_K,), jnp.float32),
            pltpu.VMEM((bpw, _CH), jnp.float32),
            pltpu.VMEM((bpw, _K), jnp.float32),
            pltpu.VMEM((bpw, _K), jnp.float32),
            pltpu.VMEM((bpw, _K), jnp.float32),
            pltpu.SemaphoreType.DMA,
        ],
        compiler_params=pltpu.CompilerParams(use_tc_tiling_on_sc=False,
                                             needs_layout_passes=False),
    )
    def gather(idx_hbm, tau_hbm, om_hbm, ap_hbm, A_hbm, P_hbm,
               ph_out, A_out, at_out,
               idx_v, idx7_v, tau_v, om_v, ap_v, A_v, P_v, at_v, sem):
        wid = lax.axis_index("s") * nc + lax.axis_index("c")
        base = wid * bpw
        pltpu.sync_copy(idx_hbm.at[pl.ds(base, bpw)], idx_v)
        pltpu.sync_copy(tau_hbm.at[pl.ds(base, bpw)], tau_v)
        pltpu.sync_copy(om_hbm, om_v)
        for t in range(bpw // nl):
            sl = pl.ds(t * nl, nl)
            idx7_v[sl] = idx_v[sl] >> 7
        copies = []
        for j in range(nch):
            sl = pl.ds(j * _CH, _CH)
            ij = idx_v.at[pl.ds(j * _CH, _CH)]
            copies.append(pltpu.async_copy(A_hbm.at[ij], A_v.at[sl], sem))
            copies.append(pltpu.async_copy(P_hbm.at[ij], P_v.at[sl], sem))
            i7 = idx7_v.at[pl.ds(j * _CH, _CH)]
            copies.append(pltpu.async_copy(ap_hbm.at[i7], ap_v.at[sl], sem))
        for c in copies:
            c.wait()
        omega = om_v[:]

        def row_block(t, _):
            for u in range(nl):
                b = t * nl + u
                bb = jnp.full((nl,), b, dtype=jnp.int32)
                tb = plsc.load_gather(tau_v, [bb])
                rv = plsc.load_gather(idx_v, [bb])
                av = plsc.load_gather(ap_v, [bb, rv & 127])
                P_v[b, :] = tb * omega + P_v[b, :]
                at_v[b, :] = av * tb
            return _

        lax.fori_loop(0, bpw // nl, row_block, None)
        fl = pl.ds(base, bpw)
        pltpu.sync_copy(P_v, ph_out.at[fl])
        pltpu.sync_copy(A_v, A_out.at[fl])
        pltpu.sync_copy(at_v, at_out.at[fl])

    return gather


def _dense_body(ph_ref, A_ref, at_ref, w_ref, b_ref, out_ref):
    zp = A_ref[:] * jnp.sin(ph_ref[:])
    zcat = jnp.concatenate([zp, at_ref[:]], axis=1)
    m = jnp.tanh(
        jnp.dot(zcat, w_ref[:], preferred_element_type=jnp.float32)
        + b_ref[:])
    cs = jnp.sum(m, axis=0, keepdims=True)
    mean = cs[:, 0:_DIM]
    for j in range(1, _PK):
        mean = mean + cs[:, j * _DIM:(j + 1) * _DIM]
    mean = mean * (1.0 / _B)
    mt = jnp.concatenate([mean] * _PK, axis=1)
    out_ref[:] = m - mt


def kernel(rel_id, tau, a_r, A_r, P_r, omega, W_proj, b_proj):
    info = plsc.get_sparse_core_info()
    gather = _make_sc_gather(info.num_cores, info.num_subcores,
                             info.num_lanes)
    a_pad = jnp.pad(a_r, (0, _APAD * 128 - a_r.shape[0])).reshape(_APAD, 128)
    ph, gA, at16 = gather(rel_id, tau, omega, a_pad, A_r, P_r)

    eye = jnp.eye(_PK, dtype=jnp.float32)
    wk = W_proj[:, 1:].T
    w0 = W_proj[:, 0]
    w_top = jnp.kron(eye, wk)
    w_bot = jnp.kron(eye, jnp.ones((_K, 1), jnp.float32) * (w0[None, :] / _K))
    w_ext = jnp.concatenate([w_top, w_bot], axis=0)
    b_tile = jnp.tile(b_proj, _PK)[None, :]

    m = pl.pallas_call(
        _dense_body,
        out_shape=jax.ShapeDtypeStruct((_M, _PK * _DIM), jnp.float32),
    )(ph.reshape(_M, 128), gA.reshape(_M, 128), at16.reshape(_M, 128),
      w_ext, b_tile)
    return m.reshape(_B, _DIM)
